# Initial kernel scaffold; baseline (speedup 1.0000x reference)
import functools
import math

import jax
import jax.numpy as jnp
from jax import lax
from jax.experimental import pallas as pl
from jax.experimental.pallas import tpu as pltpu
from jax.experimental.pallas import tpu_sc as plsc

_N = 10000
_D = 128
_E = 320000
_C = 8
_K = 5
_PAD = 2

_NC = 2
_NS = 16
_NW = _NC * _NS
_NPW = 313
_NPAD = _NW * _NPW
_CAP = 16384
_SCANCH = 6400
_NSCAN = _E // _SCANCH
_GCH = 256


def _dense_body(x_ref, alpha_ref, scale_ref, shift_ref, w_ref, b_ref, o_ref,
                *, relu):
  x = x_ref[...]
  if relu:
    x = jnp.maximum(x, 0.0)
  sqrt_n = jnp.float32(math.sqrt(_N))
  x = x - alpha_ref[...] * jnp.mean(x, axis=0, keepdims=True)
  nrm = jnp.sqrt(jnp.sum(x * x, axis=0, keepdims=True)) / sqrt_n
  x = x / nrm
  x = x * scale_ref[...] + shift_ref[...]

  n = x.shape[0]
  shifted = []
  for k in range(_K):
    s = k - _PAD
    if s < 0:
      sh = jnp.concatenate(
          [jnp.zeros((n, -s), x.dtype), x[:, : _D + s]], axis=1)
    elif s == 0:
      sh = x
    else:
      sh = jnp.concatenate(
          [x[:, s:], jnp.zeros((n, s), x.dtype)], axis=1)
    shifted.append(sh)

  w = w_ref[...]
  b = b_ref[...]
  h = None
  for c in range(_C):
    t = shifted[0] * w[c, 0]
    for k in range(1, _K):
      t = t + shifted[k] * w[c, k]
    t = t + b[0, c]
    h = t if h is None else jnp.maximum(h, t)
  o_ref[...] = h


def _dense(x, alpha, scale, shift, w, b, relu):
  return pl.pallas_call(
      functools.partial(_dense_body, relu=relu),
      out_shape=jax.ShapeDtypeStruct((_N, _D), jnp.float32),
  )(x, alpha, scale, shift, w, b)


def _readout_body(x_ref, wout_ref, bout_ref, o_ref):
  x = jnp.maximum(x_ref[...], 0.0)
  t = jnp.sum(x, axis=1, keepdims=True)
  r = lax.dot_general(wout_ref[...], t, (((1,), (0,)), ((), ())),
                      preferred_element_type=jnp.float32)
  o = jnp.zeros((8, 128), jnp.float32)
  o_ref[...] = o.at[0:3, 0:1].set(r + bout_ref[...])


def _readout(x, w_out, b_out):
  return pl.pallas_call(
      _readout_body,
      out_shape=jax.ShapeDtypeStruct((8, 128), jnp.float32),
  )(x, w_out, b_out)


def _init_acc(acc):
  neg = jnp.full((16,), -jnp.inf, jnp.float32)

  def body(i, _):
    for j in range(_D // 16):
      acc[i, pl.ds(16 * j, 16)] = neg
    return 0

  lax.fori_loop(0, _NPW, body, 0)


def _aggregate(h_hbm, slist, dlist, acc, rows, sem, cnt):
  iota = lax.broadcasted_iota(jnp.int32, (16,), 0)
  nch = (cnt + _GCH - 1) // _GCH

  def chunk(ci, _):
    base = ci * _GCH
    pltpu.async_copy(h_hbm.at[slist.at[pl.ds(base, _GCH)]], rows, sem).wait()
    ecnt = jnp.minimum(cnt - base, _GCH)

    def edge(e, _):
      t = base + e
      g = (t // 16) * 16
      lvec = dlist[pl.ds(g, 16)]
      ld = jnp.max(jnp.where(iota == (t - g), lvec, 0))
      for j in range(_D // 16):
        a = acc[ld, pl.ds(16 * j, 16)]
        r = rows[e, pl.ds(16 * j, 16)]
        acc[ld, pl.ds(16 * j, 16)] = jnp.maximum(a, r)
      return 0

    lax.fori_loop(0, ecnt, edge, 0)
    return 0

  lax.fori_loop(0, nch, chunk, 0)


def _finalize_and_store(acc, out_hbm, lo):
  big = jnp.float32(3.0e38)

  def body(i, _):
    for j in range(_D // 16):
      v = acc[i, pl.ds(16 * j, 16)]
      acc[i, pl.ds(16 * j, 16)] = jnp.where(jnp.abs(v) < big, v, 0.0)
    return 0

  lax.fori_loop(0, _NPW, body, 0)
  pltpu.sync_copy(acc, out_hbm.at[pl.ds(lo, _NPW)])


def _sc_layer1_body(src_hbm, dst_hbm, h_hbm,
                    out_hbm, slist_hbm, dlist_hbm, cnt_hbm,
                    sbuf, dbuf, slist, dlist, acc, rows, c16, sem):
  wid = lax.axis_index("s") * _NC + lax.axis_index("c")
  lo = wid * _NPW
  hi = jnp.minimum(lo + _NPW, _N)

  _init_acc(acc)

  pad = jnp.full((16,), lo, jnp.int32)

  def prefill(i, _):
    slist[pl.ds(i * 16, 16)] = pad
    return 0

  lax.fori_loop(0, _CAP // 16, prefill, 0)

  lo_v = jnp.full((16,), lo, jnp.int32)
  hi_v = jnp.full((16,), hi, jnp.int32)

  def scan_chunk(ci, wp):
    off = ci * _SCANCH
    pltpu.sync_copy(src_hbm.at[pl.ds(off, _SCANCH)], sbuf)
    pltpu.sync_copy(dst_hbm.at[pl.ds(off, _SCANCH)], dbuf)

    def scan16(i, wp):
      d16 = dbuf[pl.ds(i * 16, 16)]
      s16 = sbuf[pl.ds(i * 16, 16)]
      m = (d16 >= lo_v) & (d16 < hi_v)
      c = jnp.sum(m.astype(jnp.int32))
      plsc.store_compressed(slist.at[pl.ds(wp, 16)], s16, mask=m)
      plsc.store_compressed(dlist.at[pl.ds(wp, 16)], d16 - lo_v, mask=m)
      return jnp.minimum(wp + c, _CAP - 16)

    return lax.fori_loop(0, _SCANCH // 16, scan16, wp)

  cnt = lax.fori_loop(0, _NSCAN, scan_chunk, jnp.int32(0))

  pltpu.sync_copy(slist, slist_hbm.at[wid])
  pltpu.sync_copy(dlist, dlist_hbm.at[wid])
  c16[...] = jnp.full((16,), cnt, jnp.int32)
  pltpu.sync_copy(c16, cnt_hbm.at[wid])

  _aggregate(h_hbm, slist, dlist, acc, rows, sem, cnt)
  _finalize_and_store(acc, out_hbm, lo)


def _sc_layer2_body(slist_hbm, dlist_hbm, cnt_hbm, h_hbm,
                    out_hbm,
                    slist, dlist, acc, rows, c16, sem):
  wid = lax.axis_index("s") * _NC + lax.axis_index("c")
  lo = wid * _NPW

  _init_acc(acc)

  pltpu.sync_copy(slist_hbm.at[wid], slist)
  pltpu.sync_copy(dlist_hbm.at[wid], dlist)
  pltpu.sync_copy(cnt_hbm.at[wid], c16)
  cnt = jnp.max(c16[...])

  _aggregate(h_hbm, slist, dlist, acc, rows, sem, cnt)
  _finalize_and_store(acc, out_hbm, lo)


def _sc_mesh():
  return plsc.VectorSubcoreMesh(core_axis_name="c", subcore_axis_name="s")


_sc_layer1 = pl.kernel(
    _sc_layer1_body,
    out_type=(
        jax.ShapeDtypeStruct((_NPAD, _D), jnp.float32),
        jax.ShapeDtypeStruct((_NW, _CAP), jnp.int32),
        jax.ShapeDtypeStruct((_NW, _CAP), jnp.int32),
        jax.ShapeDtypeStruct((_NW, 16), jnp.int32),
    ),
    mesh=_sc_mesh(),
    scratch_types=(
        pltpu.VMEM((_SCANCH,), jnp.int32),
        pltpu.VMEM((_SCANCH,), jnp.int32),
        pltpu.VMEM((_CAP,), jnp.int32),
        pltpu.VMEM((_CAP,), jnp.int32),
        pltpu.VMEM((_NPW, _D), jnp.float32),
        pltpu.VMEM((_GCH, _D), jnp.float32),
        pltpu.VMEM((16,), jnp.int32),
        pltpu.SemaphoreType.DMA,
    ),
)

_sc_layer2 = pl.kernel(
    _sc_layer2_body,
    out_type=jax.ShapeDtypeStruct((_NPAD, _D), jnp.float32),
    mesh=_sc_mesh(),
    scratch_types=(
        pltpu.VMEM((_CAP,), jnp.int32),
        pltpu.VMEM((_CAP,), jnp.int32),
        pltpu.VMEM((_NPW, _D), jnp.float32),
        pltpu.VMEM((_GCH, _D), jnp.float32),
        pltpu.VMEM((16,), jnp.int32),
        pltpu.SemaphoreType.DMA,
    ),
)


def kernel(x, edge_index, W1, b1, W2, b2,
           alpha0, scale0, shift0, alpha1, scale1, shift1,
           W_out, b_out):
  src = edge_index[0]
  dst = edge_index[1]
  w1 = W1.reshape(_C, _K)
  w2 = W2.reshape(_C, _K)
  b1r = b1.reshape(1, _C)
  b2r = b2.reshape(1, _C)
  al0 = alpha0.reshape(1, _D)
  sc0 = scale0.reshape(1, _D)
  sh0 = shift0.reshape(1, _D)
  al1 = alpha1.reshape(1, _D)
  sc1 = scale1.reshape(1, _D)
  sh1 = shift1.reshape(1, _D)

  h0 = _dense(x, al0, sc0, sh0, w1, b1r, relu=False)
  out0p, slist, dlist, cnts = _sc_layer1(src, dst, h0)
  h1 = _dense(out0p[:_N], al1, sc1, sh1, w2, b2r, relu=True)
  out1p = _sc_layer2(slist, dlist, cnts, h1)
  o = _readout(out1p[:_N], W_out, b_out.reshape(3, 1))
  return o[0:3, 0]


# speedup vs baseline: 2.1421x; 2.1421x over previous
import functools
import math

import jax
import jax.numpy as jnp
from jax import lax
from jax.experimental import pallas as pl
from jax.experimental.pallas import tpu as pltpu
from jax.experimental.pallas import tpu_sc as plsc

_N = 10000
_D = 128
_E = 320000
_C = 8
_K = 5
_PAD = 2

_NC = 2
_NS = 16
_NW = _NC * _NS
_NPW = 320
_NPAD = _NW * _NPW
_CAP = 16384
_SCANCH = 6400
_NSCAN = _E // _SCANCH
_GCH = 256


def _dense_body(x_ref, alpha_ref, scale_ref, shift_ref, w_ref, b_ref, o_ref,
                *, relu):
  x = x_ref[...]
  if relu:
    x = jnp.maximum(x, 0.0)
  sqrt_n = jnp.float32(math.sqrt(_N))
  x = x - alpha_ref[...] * jnp.mean(x, axis=0, keepdims=True)
  nrm = jnp.sqrt(jnp.sum(x * x, axis=0, keepdims=True)) / sqrt_n
  x = x / nrm
  x = x * scale_ref[...] + shift_ref[...]

  n = x.shape[0]
  shifted = []
  for k in range(_K):
    s = k - _PAD
    if s < 0:
      sh = jnp.concatenate(
          [jnp.zeros((n, -s), x.dtype), x[:, : _D + s]], axis=1)
    elif s == 0:
      sh = x
    else:
      sh = jnp.concatenate(
          [x[:, s:], jnp.zeros((n, s), x.dtype)], axis=1)
    shifted.append(sh)

  w = w_ref[...]
  b = b_ref[...]
  h = None
  for c in range(_C):
    t = shifted[0] * w[c, 0]
    for k in range(1, _K):
      t = t + shifted[k] * w[c, k]
    t = t + b[0, c]
    h = t if h is None else jnp.maximum(h, t)
  o_ref[...] = h


def _dense(x, alpha, scale, shift, w, b, relu):
  return pl.pallas_call(
      functools.partial(_dense_body, relu=relu),
      out_shape=jax.ShapeDtypeStruct((_N, _D), jnp.float32),
  )(x, alpha, scale, shift, w, b)


def _readout_body(x_ref, wout_ref, bout_ref, o_ref):
  x = jnp.maximum(x_ref[...], 0.0)
  t = jnp.sum(x, axis=1, keepdims=True)
  r = lax.dot_general(wout_ref[...], t, (((1,), (0,)), ((), ())),
                      preferred_element_type=jnp.float32)
  rp = jnp.concatenate([r + bout_ref[...], jnp.zeros((5, 1), jnp.float32)],
                       axis=0)
  o_ref[...] = rp * jnp.ones((8, 128), jnp.float32)


def _readout(x, w_out, b_out):
  return pl.pallas_call(
      _readout_body,
      out_shape=jax.ShapeDtypeStruct((8, 128), jnp.float32),
  )(x, w_out, b_out)


def _init_acc(acc):
  neg = jnp.full((16,), -jnp.inf, jnp.float32)

  def body(i, _):
    for j in range(_D // 16):
      acc[i, pl.ds(16 * j, 16)] = neg
    return 0

  lax.fori_loop(0, _NPW, body, 0)


def _aggregate(h_hbm, slist, dlist, acc, rows, sem, cnt):
  iota = lax.broadcasted_iota(jnp.int32, (16,), 0)
  nch = (cnt + _GCH - 1) // _GCH

  def chunk(ci, _):
    base = ci * _GCH
    pltpu.async_copy(h_hbm.at[slist.at[pl.ds(base, _GCH)]], rows, sem).wait()
    ecnt = jnp.minimum(cnt - base, _GCH)

    def edge(e, _):
      t = base + e
      g = (t // 16) * 16
      lvec = dlist[pl.ds(g, 16)]
      ld = jnp.max(jnp.where(iota == (t - g), lvec, 0))
      for j in range(_D // 16):
        a = acc[ld, pl.ds(16 * j, 16)]
        r = rows[e, pl.ds(16 * j, 16)]
        acc[ld, pl.ds(16 * j, 16)] = jnp.maximum(a, r)
      return 0

    lax.fori_loop(0, ecnt, edge, 0)
    return 0

  lax.fori_loop(0, nch, chunk, 0)


def _finalize_and_store(acc, out_hbm, lo):
  big = jnp.float32(3.0e38)

  def body(i, _):
    for j in range(_D // 16):
      v = acc[i, pl.ds(16 * j, 16)]
      acc[i, pl.ds(16 * j, 16)] = jnp.where(jnp.abs(v) < big, v, 0.0)
    return 0

  lax.fori_loop(0, _NPW, body, 0)
  pltpu.sync_copy(acc, out_hbm.at[pl.ds(lo, _NPW)])


def _sc_layer1_body(src_hbm, dst_hbm, h_hbm,
                    out_hbm, slist_hbm, dlist_hbm, cnt_hbm,
                    sbuf, dbuf, slist, dlist, acc, rows, c16, sem):
  wid = lax.axis_index("s") * _NC + lax.axis_index("c")
  lo = wid * _NPW
  hi = jnp.minimum(lo + _NPW, _N)

  _init_acc(acc)

  pad = jnp.full((16,), lo, jnp.int32)

  def prefill(i, _):
    slist[pl.ds(i * 16, 16)] = pad
    return 0

  lax.fori_loop(0, _CAP // 16, prefill, 0)

  lo_v = jnp.full((16,), lo, jnp.int32)
  hi_v = jnp.full((16,), hi, jnp.int32)

  def scan_chunk(ci, wp):
    off = ci * _SCANCH
    pltpu.sync_copy(src_hbm.at[pl.ds(off, _SCANCH)], sbuf)
    pltpu.sync_copy(dst_hbm.at[pl.ds(off, _SCANCH)], dbuf)

    def scan16(i, wp):
      d16 = dbuf[pl.ds(i * 16, 16)]
      s16 = sbuf[pl.ds(i * 16, 16)]
      m = (d16 >= lo_v) & (d16 < hi_v)
      c = jnp.sum(m.astype(jnp.int32))
      plsc.store_compressed(slist.at[pl.ds(wp, 16)], s16, mask=m)
      plsc.store_compressed(dlist.at[pl.ds(wp, 16)], d16 - lo_v, mask=m)
      return jnp.minimum(wp + c, _CAP - 16)

    return lax.fori_loop(0, _SCANCH // 16, scan16, wp)

  cnt = lax.fori_loop(0, _NSCAN, scan_chunk, jnp.int32(0))

  pltpu.sync_copy(slist, slist_hbm.at[wid])
  pltpu.sync_copy(dlist, dlist_hbm.at[wid])
  c16[...] = jnp.full((16,), cnt, jnp.int32)
  pltpu.sync_copy(c16, cnt_hbm.at[wid])

  _aggregate(h_hbm, slist, dlist, acc, rows, sem, cnt)
  _finalize_and_store(acc, out_hbm, lo)


def _sc_layer2_body(slist_hbm, dlist_hbm, cnt_hbm, h_hbm,
                    out_hbm,
                    slist, dlist, acc, rows, c16, sem):
  wid = lax.axis_index("s") * _NC + lax.axis_index("c")
  lo = wid * _NPW

  _init_acc(acc)

  pltpu.sync_copy(slist_hbm.at[wid], slist)
  pltpu.sync_copy(dlist_hbm.at[wid], dlist)
  pltpu.sync_copy(cnt_hbm.at[wid], c16)
  cnt = jnp.max(c16[...])

  _aggregate(h_hbm, slist, dlist, acc, rows, sem, cnt)
  _finalize_and_store(acc, out_hbm, lo)


def _sc_mesh():
  return plsc.VectorSubcoreMesh(core_axis_name="c", subcore_axis_name="s")


_sc_layer1 = pl.kernel(
    _sc_layer1_body,
    out_type=(
        jax.ShapeDtypeStruct((_NPAD, _D), jnp.float32),
        jax.ShapeDtypeStruct((_NW, _CAP), jnp.int32),
        jax.ShapeDtypeStruct((_NW, _CAP), jnp.int32),
        jax.ShapeDtypeStruct((_NW, 16), jnp.int32),
    ),
    mesh=_sc_mesh(),
    compiler_params=pltpu.CompilerParams(needs_layout_passes=False),
    scratch_types=(
        pltpu.VMEM((_SCANCH,), jnp.int32),
        pltpu.VMEM((_SCANCH,), jnp.int32),
        pltpu.VMEM((_CAP,), jnp.int32),
        pltpu.VMEM((_CAP,), jnp.int32),
        pltpu.VMEM((_NPW, _D), jnp.float32),
        pltpu.VMEM((_GCH, _D), jnp.float32),
        pltpu.VMEM((16,), jnp.int32),
        pltpu.SemaphoreType.DMA,
    ),
)

_sc_layer2 = pl.kernel(
    _sc_layer2_body,
    out_type=jax.ShapeDtypeStruct((_NPAD, _D), jnp.float32),
    mesh=_sc_mesh(),
    compiler_params=pltpu.CompilerParams(needs_layout_passes=False),
    scratch_types=(
        pltpu.VMEM((_CAP,), jnp.int32),
        pltpu.VMEM((_CAP,), jnp.int32),
        pltpu.VMEM((_NPW, _D), jnp.float32),
        pltpu.VMEM((_GCH, _D), jnp.float32),
        pltpu.VMEM((16,), jnp.int32),
        pltpu.SemaphoreType.DMA,
    ),
)


def kernel(x, edge_index, W1, b1, W2, b2,
           alpha0, scale0, shift0, alpha1, scale1, shift1,
           W_out, b_out):
  src = edge_index[0]
  dst = edge_index[1]
  w1 = W1.reshape(_C, _K)
  w2 = W2.reshape(_C, _K)
  b1r = b1.reshape(1, _C)
  b2r = b2.reshape(1, _C)
  al0 = alpha0.reshape(1, _D)
  sc0 = scale0.reshape(1, _D)
  sh0 = shift0.reshape(1, _D)
  al1 = alpha1.reshape(1, _D)
  sc1 = scale1.reshape(1, _D)
  sh1 = shift1.reshape(1, _D)

  h0 = _dense(x, al0, sc0, sh0, w1, b1r, relu=False)
  out0p, slist, dlist, cnts = _sc_layer1(src, dst, h0)
  h1 = _dense(out0p[:_N], al1, sc1, sh1, w2, b2r, relu=True)
  out1p = _sc_layer2(slist, dlist, cnts, h1)
  o = _readout(out1p[:_N], W_out, b_out.reshape(3, 1))
  return o[0:3, 0]


# speedup vs baseline: 2.6948x; 1.2580x over previous
import functools
import math

import jax
import jax.numpy as jnp
from jax import lax
from jax.experimental import pallas as pl
from jax.experimental.pallas import tpu as pltpu
from jax.experimental.pallas import tpu_sc as plsc

_N = 10000
_D = 128
_E = 320000
_C = 8
_K = 5
_PAD = 2

_NC = 2
_NS = 16
_NW = _NC * _NS
_NPW = 320
_NPAD = _NW * _NPW
_CAP = 16384
_SCANCH = 6400
_NSCAN = _E // _SCANCH
_GCH = 256
_NACC = _NPW + 16


def _dense_body(x_ref, alpha_ref, scale_ref, shift_ref, w_ref, b_ref, o_ref,
                *, relu):
  x = x_ref[...]
  if relu:
    x = jnp.maximum(x, 0.0)
  sqrt_n = jnp.float32(math.sqrt(_N))
  x = x - alpha_ref[...] * jnp.mean(x, axis=0, keepdims=True)
  nrm = jnp.sqrt(jnp.sum(x * x, axis=0, keepdims=True)) / sqrt_n
  x = x / nrm
  x = x * scale_ref[...] + shift_ref[...]

  n = x.shape[0]
  shifted = []
  for k in range(_K):
    s = k - _PAD
    if s < 0:
      sh = jnp.concatenate(
          [jnp.zeros((n, -s), x.dtype), x[:, : _D + s]], axis=1)
    elif s == 0:
      sh = x
    else:
      sh = jnp.concatenate(
          [x[:, s:], jnp.zeros((n, s), x.dtype)], axis=1)
    shifted.append(sh)

  w = w_ref[...]
  b = b_ref[...]
  h = None
  for c in range(_C):
    t = shifted[0] * w[c, 0]
    for k in range(1, _K):
      t = t + shifted[k] * w[c, k]
    t = t + b[0, c]
    h = t if h is None else jnp.maximum(h, t)
  o_ref[...] = h


def _dense(x, alpha, scale, shift, w, b, relu):
  return pl.pallas_call(
      functools.partial(_dense_body, relu=relu),
      out_shape=jax.ShapeDtypeStruct((_N, _D), jnp.float32),
  )(x, alpha, scale, shift, w, b)


def _readout_body(x_ref, wout_ref, bout_ref, o_ref):
  x = jnp.maximum(x_ref[...], 0.0)
  t = jnp.sum(x, axis=1, keepdims=True)
  r = lax.dot_general(wout_ref[...], t, (((1,), (0,)), ((), ())),
                      preferred_element_type=jnp.float32)
  rp = jnp.concatenate([r + bout_ref[...], jnp.zeros((5, 1), jnp.float32)],
                       axis=0)
  o_ref[...] = rp * jnp.ones((8, 128), jnp.float32)


def _readout(x, w_out, b_out):
  return pl.pallas_call(
      _readout_body,
      out_shape=jax.ShapeDtypeStruct((8, 128), jnp.float32),
  )(x, w_out, b_out)


def _init_acc(acc):
  neg = jnp.full((16,), -jnp.inf, jnp.float32)

  def body(i, _):
    for j in range(_D // 16):
      acc[i, pl.ds(16 * j, 16)] = neg
    return 0

  lax.fori_loop(0, _NACC, body, 0)


def _aggregate(h_hbm, slist, dlist, acc, rows, sem, cnt):
  nch = (cnt + _GCH - 1) // _GCH

  def chunk(ci, _):
    base = ci * _GCH
    pltpu.async_copy(h_hbm.at[slist.at[pl.ds(base, _GCH)]], rows, sem).wait()

    def group(g, _):
      lvec = dlist[pl.ds(base + g * 16, 16)]
      for lane in range(16):
        ld = lvec[lane]
        e = g * 16 + lane
        for j in range(_D // 16):
          a = acc[ld, pl.ds(16 * j, 16)]
          r = rows[e, pl.ds(16 * j, 16)]
          acc[ld, pl.ds(16 * j, 16)] = jnp.maximum(a, r)
      return 0

    lax.fori_loop(0, _GCH // 16, group, 0)
    return 0

  lax.fori_loop(0, nch, chunk, 0)


def _finalize_and_store(acc, out_hbm, lo):
  big = jnp.float32(3.0e38)

  def body(i, _):
    for j in range(_D // 16):
      v = acc[i, pl.ds(16 * j, 16)]
      acc[i, pl.ds(16 * j, 16)] = jnp.where(jnp.abs(v) < big, v, 0.0)
    return 0

  lax.fori_loop(0, _NPW, body, 0)
  pltpu.sync_copy(acc.at[pl.ds(0, _NPW)], out_hbm.at[pl.ds(lo, _NPW)])


def _sc_layer1_body(src_hbm, dst_hbm, h_hbm,
                    out_hbm, slist_hbm, dlist_hbm, cnt_hbm,
                    sbuf, dbuf, slist, dlist, acc, rows, c16, sem):
  wid = lax.axis_index("s") * _NC + lax.axis_index("c")
  lo = wid * _NPW
  hi = jnp.minimum(lo + _NPW, _N)

  _init_acc(acc)

  pad_s = jnp.full((16,), lo, jnp.int32)
  pad_d = jnp.full((16,), _NPW, jnp.int32)

  def prefill(i, _):
    slist[pl.ds(i * 16, 16)] = pad_s
    dlist[pl.ds(i * 16, 16)] = pad_d
    return 0

  lax.fori_loop(0, _CAP // 16, prefill, 0)

  lo_v = jnp.full((16,), lo, jnp.int32)
  hi_v = jnp.full((16,), hi, jnp.int32)

  def scan_chunk(ci, wp):
    off = ci * _SCANCH
    pltpu.sync_copy(src_hbm.at[pl.ds(off, _SCANCH)], sbuf)
    pltpu.sync_copy(dst_hbm.at[pl.ds(off, _SCANCH)], dbuf)

    def scan16(i, wp):
      d16 = dbuf[pl.ds(i * 16, 16)]
      s16 = sbuf[pl.ds(i * 16, 16)]
      m = (d16 >= lo_v) & (d16 < hi_v)
      c = plsc.all_reduce_population_count(m)[0]
      plsc.store_compressed(slist.at[pl.ds(wp, 16)], s16, mask=m)
      plsc.store_compressed(dlist.at[pl.ds(wp, 16)], d16 - lo_v, mask=m)
      return jnp.minimum(wp + c, _CAP - 16)

    return lax.fori_loop(0, _SCANCH // 16, scan16, wp)

  cnt = lax.fori_loop(0, _NSCAN, scan_chunk, jnp.int32(0))

  pltpu.sync_copy(slist, slist_hbm.at[wid])
  pltpu.sync_copy(dlist, dlist_hbm.at[wid])
  c16[...] = jnp.full((16,), cnt, jnp.int32)
  pltpu.sync_copy(c16, cnt_hbm.at[wid])

  _aggregate(h_hbm, slist, dlist, acc, rows, sem, cnt)
  _finalize_and_store(acc, out_hbm, lo)


def _sc_layer2_body(slist_hbm, dlist_hbm, cnt_hbm, h_hbm,
                    out_hbm,
                    slist, dlist, acc, rows, c16, sem):
  wid = lax.axis_index("s") * _NC + lax.axis_index("c")
  lo = wid * _NPW

  _init_acc(acc)

  pltpu.sync_copy(slist_hbm.at[wid], slist)
  pltpu.sync_copy(dlist_hbm.at[wid], dlist)
  pltpu.sync_copy(cnt_hbm.at[wid], c16)
  cnt = jnp.max(c16[...])

  _aggregate(h_hbm, slist, dlist, acc, rows, sem, cnt)
  _finalize_and_store(acc, out_hbm, lo)


def _sc_mesh():
  return plsc.VectorSubcoreMesh(core_axis_name="c", subcore_axis_name="s")


_sc_layer1 = pl.kernel(
    _sc_layer1_body,
    out_type=(
        jax.ShapeDtypeStruct((_NPAD, _D), jnp.float32),
        jax.ShapeDtypeStruct((_NW, _CAP), jnp.int32),
        jax.ShapeDtypeStruct((_NW, _CAP), jnp.int32),
        jax.ShapeDtypeStruct((_NW, 16), jnp.int32),
    ),
    mesh=_sc_mesh(),
    compiler_params=pltpu.CompilerParams(needs_layout_passes=False),
    scratch_types=(
        pltpu.VMEM((_SCANCH,), jnp.int32),
        pltpu.VMEM((_SCANCH,), jnp.int32),
        pltpu.VMEM((_CAP,), jnp.int32),
        pltpu.VMEM((_CAP,), jnp.int32),
        pltpu.VMEM((_NACC, _D), jnp.float32),
        pltpu.VMEM((_GCH, _D), jnp.float32),
        pltpu.VMEM((16,), jnp.int32),
        pltpu.SemaphoreType.DMA,
    ),
)

_sc_layer2 = pl.kernel(
    _sc_layer2_body,
    out_type=jax.ShapeDtypeStruct((_NPAD, _D), jnp.float32),
    mesh=_sc_mesh(),
    compiler_params=pltpu.CompilerParams(needs_layout_passes=False),
    scratch_types=(
        pltpu.VMEM((_CAP,), jnp.int32),
        pltpu.VMEM((_CAP,), jnp.int32),
        pltpu.VMEM((_NACC, _D), jnp.float32),
        pltpu.VMEM((_GCH, _D), jnp.float32),
        pltpu.VMEM((16,), jnp.int32),
        pltpu.SemaphoreType.DMA,
    ),
)


def kernel(x, edge_index, W1, b1, W2, b2,
           alpha0, scale0, shift0, alpha1, scale1, shift1,
           W_out, b_out):
  src = edge_index[0]
  dst = edge_index[1]
  w1 = W1.reshape(_C, _K)
  w2 = W2.reshape(_C, _K)
  b1r = b1.reshape(1, _C)
  b2r = b2.reshape(1, _C)
  al0 = alpha0.reshape(1, _D)
  sc0 = scale0.reshape(1, _D)
  sh0 = shift0.reshape(1, _D)
  al1 = alpha1.reshape(1, _D)
  sc1 = scale1.reshape(1, _D)
  sh1 = shift1.reshape(1, _D)

  h0 = _dense(x, al0, sc0, sh0, w1, b1r, relu=False)
  out0p, slist, dlist, cnts = _sc_layer1(src, dst, h0)
  h1 = _dense(out0p[:_N], al1, sc1, sh1, w2, b2r, relu=True)
  out1p = _sc_layer2(slist, dlist, cnts, h1)
  o = _readout(out1p[:_N], W_out, b_out.reshape(3, 1))
  return o[0:3, 0]


# speedup vs baseline: 3.1087x; 1.1536x over previous
import functools
import math

import jax
import jax.numpy as jnp
from jax import lax
from jax.experimental import pallas as pl
from jax.experimental.pallas import tpu as pltpu
from jax.experimental.pallas import tpu_sc as plsc

_N = 10000
_D = 128
_E = 320000
_C = 8
_K = 5
_PAD = 2

_NC = 2
_NS = 16
_NW = _NC * _NS
_NPW = 320
_NPAD = _NW * _NPW
_CAP = 16384
_SCANCH = 6400
_NSCAN = _E // _SCANCH
_GCH = 128
_NACC = _NPW + 16


def _dense_body(x_ref, alpha_ref, scale_ref, shift_ref, w_ref, b_ref, o_ref,
                *, relu):
  x = x_ref[...]
  if relu:
    x = jnp.maximum(x, 0.0)
  sqrt_n = jnp.float32(math.sqrt(_N))
  x = x - alpha_ref[...] * jnp.mean(x, axis=0, keepdims=True)
  nrm = jnp.sqrt(jnp.sum(x * x, axis=0, keepdims=True)) / sqrt_n
  x = x / nrm
  x = x * scale_ref[...] + shift_ref[...]

  n = x.shape[0]
  shifted = []
  for k in range(_K):
    s = k - _PAD
    if s < 0:
      sh = jnp.concatenate(
          [jnp.zeros((n, -s), x.dtype), x[:, : _D + s]], axis=1)
    elif s == 0:
      sh = x
    else:
      sh = jnp.concatenate(
          [x[:, s:], jnp.zeros((n, s), x.dtype)], axis=1)
    shifted.append(sh)

  w = w_ref[...]
  b = b_ref[...]
  h = None
  for c in range(_C):
    t = shifted[0] * w[c, 0]
    for k in range(1, _K):
      t = t + shifted[k] * w[c, k]
    t = t + b[0, c]
    h = t if h is None else jnp.maximum(h, t)
  o_ref[...] = h


def _dense(x, alpha, scale, shift, w, b, relu):
  return pl.pallas_call(
      functools.partial(_dense_body, relu=relu),
      out_shape=jax.ShapeDtypeStruct((_N, _D), jnp.float32),
  )(x, alpha, scale, shift, w, b)


def _readout_body(x_ref, wout_ref, bout_ref, o_ref):
  x = jnp.maximum(x_ref[...], 0.0)
  t = jnp.sum(x, axis=1, keepdims=True)
  r = lax.dot_general(wout_ref[...], t, (((1,), (0,)), ((), ())),
                      preferred_element_type=jnp.float32)
  rp = jnp.concatenate([r + bout_ref[...], jnp.zeros((5, 1), jnp.float32)],
                       axis=0)
  o_ref[...] = rp * jnp.ones((8, 128), jnp.float32)


def _readout(x, w_out, b_out):
  return pl.pallas_call(
      _readout_body,
      out_shape=jax.ShapeDtypeStruct((8, 128), jnp.float32),
  )(x, w_out, b_out)


def _init_acc(acc):
  neg = jnp.full((16,), -jnp.inf, jnp.float32)

  def body(i, _):
    for j in range(_D // 16):
      acc[i, pl.ds(16 * j, 16)] = neg
    return 0

  lax.fori_loop(0, _NACC, body, 0)


def _aggregate(h_hbm, slist, dlist, acc, rows0, rows1, sem0, sem1, cnt):
  nch = (cnt + _GCH - 1) // _GCH
  bufs = ((rows0, sem0), (rows1, sem1))

  def start(ci, b):
    rbuf, sem = bufs[b]
    pltpu.async_copy(h_hbm.at[slist.at[pl.ds(ci * _GCH, _GCH)]], rbuf, sem)

  def wait(b):
    rbuf, sem = bufs[b]
    pltpu.make_async_copy(h_hbm.at[pl.ds(0, _GCH)], rbuf, sem).wait()

  def process(base, b):
    rbuf, _ = bufs[b]

    def group(g, _):
      lvec = dlist[pl.ds(base + g * 16, 16)]
      for lane in range(16):
        ld = lvec[lane]
        e = g * 16 + lane
        for j in range(_D // 16):
          a = acc[ld, pl.ds(16 * j, 16)]
          r = rbuf[e, pl.ds(16 * j, 16)]
          acc[ld, pl.ds(16 * j, 16)] = jnp.maximum(a, r)
      return 0

    lax.fori_loop(0, _GCH // 16, group, 0)

  @pl.when(nch > 0)
  def _():
    start(0, 0)

    def pair(p, _):
      c0 = 2 * p
      c1 = c0 + 1
      wait(0)

      @pl.when(c1 < nch)
      def _():
        start(c1, 1)

      process(c0 * _GCH, 0)

      @pl.when(c1 < nch)
      def _():
        wait(1)

        @pl.when(c1 + 1 < nch)
        def _():
          start(c1 + 1, 0)

        process(c1 * _GCH, 1)

      return 0

    lax.fori_loop(0, (nch + 1) // 2, pair, 0)


def _finalize_and_store(acc, out_hbm, lo):
  big = jnp.float32(3.0e38)

  def body(i, _):
    for j in range(_D // 16):
      v = acc[i, pl.ds(16 * j, 16)]
      acc[i, pl.ds(16 * j, 16)] = jnp.where(jnp.abs(v) < big, v, 0.0)
    return 0

  lax.fori_loop(0, _NPW, body, 0)
  pltpu.sync_copy(acc.at[pl.ds(0, _NPW)], out_hbm.at[pl.ds(lo, _NPW)])


def _sc_layer1_body(src_hbm, dst_hbm, h_hbm,
                    out_hbm, slist_hbm, dlist_hbm, cnt_hbm,
                    sbuf, dbuf, slist, dlist, acc, rows0, rows1, c16,
                    sem0, sem1):
  wid = lax.axis_index("s") * _NC + lax.axis_index("c")
  lo = wid * _NPW
  hi = jnp.minimum(lo + _NPW, _N)

  _init_acc(acc)

  pad_s = jnp.full((16,), lo, jnp.int32)
  pad_d = jnp.full((16,), _NPW, jnp.int32)

  def prefill(i, _):
    slist[pl.ds(i * 16, 16)] = pad_s
    dlist[pl.ds(i * 16, 16)] = pad_d
    return 0

  lax.fori_loop(0, _CAP // 16, prefill, 0)

  lo_v = jnp.full((16,), lo, jnp.int32)
  hi_v = jnp.full((16,), hi, jnp.int32)

  def scan_chunk(ci, wp):
    off = ci * _SCANCH
    pltpu.sync_copy(src_hbm.at[pl.ds(off, _SCANCH)], sbuf)
    pltpu.sync_copy(dst_hbm.at[pl.ds(off, _SCANCH)], dbuf)

    def scan16(i, wp):
      d16 = dbuf[pl.ds(i * 16, 16)]
      s16 = sbuf[pl.ds(i * 16, 16)]
      m = (d16 >= lo_v) & (d16 < hi_v)
      c = plsc.all_reduce_population_count(m)[0]
      plsc.store_compressed(slist.at[pl.ds(wp, 16)], s16, mask=m)
      plsc.store_compressed(dlist.at[pl.ds(wp, 16)], d16 - lo_v, mask=m)
      return jnp.minimum(wp + c, _CAP - 16)

    return lax.fori_loop(0, _SCANCH // 16, scan16, wp, unroll=4)

  cnt = lax.fori_loop(0, _NSCAN, scan_chunk, jnp.int32(0))

  pltpu.sync_copy(slist, slist_hbm.at[wid])
  pltpu.sync_copy(dlist, dlist_hbm.at[wid])
  c16[...] = jnp.full((16,), cnt, jnp.int32)
  pltpu.sync_copy(c16, cnt_hbm.at[wid])

  _aggregate(h_hbm, slist, dlist, acc, rows0, rows1, sem0, sem1, cnt)
  _finalize_and_store(acc, out_hbm, lo)


def _sc_layer2_body(slist_hbm, dlist_hbm, cnt_hbm, h_hbm,
                    out_hbm,
                    slist, dlist, acc, rows0, rows1, c16, sem0, sem1):
  wid = lax.axis_index("s") * _NC + lax.axis_index("c")
  lo = wid * _NPW

  _init_acc(acc)

  pltpu.sync_copy(slist_hbm.at[wid], slist)
  pltpu.sync_copy(dlist_hbm.at[wid], dlist)
  pltpu.sync_copy(cnt_hbm.at[wid], c16)
  cnt = jnp.max(c16[...])

  _aggregate(h_hbm, slist, dlist, acc, rows0, rows1, sem0, sem1, cnt)
  _finalize_and_store(acc, out_hbm, lo)


def _sc_mesh():
  return plsc.VectorSubcoreMesh(core_axis_name="c", subcore_axis_name="s")


_sc_layer1 = pl.kernel(
    _sc_layer1_body,
    out_type=(
        jax.ShapeDtypeStruct((_NPAD, _D), jnp.float32),
        jax.ShapeDtypeStruct((_NW, _CAP), jnp.int32),
        jax.ShapeDtypeStruct((_NW, _CAP), jnp.int32),
        jax.ShapeDtypeStruct((_NW, 16), jnp.int32),
    ),
    mesh=_sc_mesh(),
    compiler_params=pltpu.CompilerParams(needs_layout_passes=False),
    scratch_types=(
        pltpu.VMEM((_SCANCH,), jnp.int32),
        pltpu.VMEM((_SCANCH,), jnp.int32),
        pltpu.VMEM((_CAP,), jnp.int32),
        pltpu.VMEM((_CAP,), jnp.int32),
        pltpu.VMEM((_NACC, _D), jnp.float32),
        pltpu.VMEM((_GCH, _D), jnp.float32),
        pltpu.VMEM((_GCH, _D), jnp.float32),
        pltpu.VMEM((16,), jnp.int32),
        pltpu.SemaphoreType.DMA,
        pltpu.SemaphoreType.DMA,
    ),
)

_sc_layer2 = pl.kernel(
    _sc_layer2_body,
    out_type=jax.ShapeDtypeStruct((_NPAD, _D), jnp.float32),
    mesh=_sc_mesh(),
    compiler_params=pltpu.CompilerParams(needs_layout_passes=False),
    scratch_types=(
        pltpu.VMEM((_CAP,), jnp.int32),
        pltpu.VMEM((_CAP,), jnp.int32),
        pltpu.VMEM((_NACC, _D), jnp.float32),
        pltpu.VMEM((_GCH, _D), jnp.float32),
        pltpu.VMEM((_GCH, _D), jnp.float32),
        pltpu.VMEM((16,), jnp.int32),
        pltpu.SemaphoreType.DMA,
        pltpu.SemaphoreType.DMA,
    ),
)


def kernel(x, edge_index, W1, b1, W2, b2,
           alpha0, scale0, shift0, alpha1, scale1, shift1,
           W_out, b_out):
  src = edge_index[0]
  dst = edge_index[1]
  w1 = W1.reshape(_C, _K)
  w2 = W2.reshape(_C, _K)
  b1r = b1.reshape(1, _C)
  b2r = b2.reshape(1, _C)
  al0 = alpha0.reshape(1, _D)
  sc0 = scale0.reshape(1, _D)
  sh0 = shift0.reshape(1, _D)
  al1 = alpha1.reshape(1, _D)
  sc1 = scale1.reshape(1, _D)
  sh1 = shift1.reshape(1, _D)

  h0 = _dense(x, al0, sc0, sh0, w1, b1r, relu=False)
  out0p, slist, dlist, cnts = _sc_layer1(src, dst, h0)
  h1 = _dense(out0p[:_N], al1, sc1, sh1, w2, b2r, relu=True)
  out1p = _sc_layer2(slist, dlist, cnts, h1)
  o = _readout(out1p[:_N], W_out, b_out.reshape(3, 1))
  return o[0:3, 0]


# speedup vs baseline: 4.5699x; 1.4700x over previous
import functools
import math

import jax
import jax.numpy as jnp
from jax import lax
from jax.experimental import pallas as pl
from jax.experimental.pallas import tpu as pltpu
from jax.experimental.pallas import tpu_sc as plsc

_N = 10000
_D = 128
_E = 320000
_C = 8
_K = 5
_PAD = 2

_NC = 2
_NS = 16
_NW = _NC * _NS
_NPW = 320
_NPAD = _NW * _NPW
_CAP = 12800
_SCANCH = 4000
_NSCAN = _E // _SCANCH
_GCH = 128
_NACC = _NPW + 16
_NSM = 352


def _dense_body(x_ref, alpha_ref, scale_ref, shift_ref, w_ref, b_ref, o_ref,
                *, relu):
  x = x_ref[...]
  if relu:
    x = jnp.maximum(x, 0.0)
  sqrt_n = jnp.float32(math.sqrt(_N))
  x = x - alpha_ref[...] * jnp.mean(x, axis=0, keepdims=True)
  nrm = jnp.sqrt(jnp.sum(x * x, axis=0, keepdims=True)) / sqrt_n
  x = x / nrm
  x = x * scale_ref[...] + shift_ref[...]

  n = x.shape[0]
  shifted = []
  for k in range(_K):
    s = k - _PAD
    if s < 0:
      sh = jnp.concatenate(
          [jnp.zeros((n, -s), x.dtype), x[:, : _D + s]], axis=1)
    elif s == 0:
      sh = x
    else:
      sh = jnp.concatenate(
          [x[:, s:], jnp.zeros((n, s), x.dtype)], axis=1)
    shifted.append(sh)

  w = w_ref[...]
  b = b_ref[...]
  h = None
  for c in range(_C):
    t = shifted[0] * w[c, 0]
    for k in range(1, _K):
      t = t + shifted[k] * w[c, k]
    t = t + b[0, c]
    h = t if h is None else jnp.maximum(h, t)
  o_ref[...] = h


def _dense(x, alpha, scale, shift, w, b, relu):
  return pl.pallas_call(
      functools.partial(_dense_body, relu=relu),
      out_shape=jax.ShapeDtypeStruct((_N, _D), jnp.float32),
  )(x, alpha, scale, shift, w, b)


def _readout_body(x_ref, wout_ref, bout_ref, o_ref):
  x = jnp.maximum(x_ref[...], 0.0)
  t = jnp.sum(x, axis=1, keepdims=True)
  r = lax.dot_general(wout_ref[...], t, (((1,), (0,)), ((), ())),
                      preferred_element_type=jnp.float32)
  rp = jnp.concatenate([r + bout_ref[...], jnp.zeros((5, 1), jnp.float32)],
                       axis=0)
  o_ref[...] = rp * jnp.ones((8, 128), jnp.float32)


def _readout(x, w_out, b_out):
  return pl.pallas_call(
      _readout_body,
      out_shape=jax.ShapeDtypeStruct((8, 128), jnp.float32),
  )(x, w_out, b_out)


def _init_acc(acc):
  neg = jnp.full((16,), -jnp.inf, jnp.float32)

  def body(i, _):
    for j in range(_D // 16):
      acc[i, pl.ds(16 * j, 16)] = neg
    return 0

  lax.fori_loop(0, _NACC, body, 0)


def _sort_edges(slist, dlist, ssort, pos_smem, off_smem, cnt):
  iota = lax.broadcasted_iota(jnp.int32, (16,), 0)
  nvr = (cnt + 15) // 16

  def zero(i, _):
    pos_smem[i] = 0
    return 0

  lax.fori_loop(0, _NSM, zero, 0)

  def hist(g, _):
    lvec = dlist[pl.ds(g * 16, 16)]
    for lane in range(16):
      ld = lvec[lane]
      pos_smem[ld] = pos_smem[ld] + 1
    return 0

  lax.fori_loop(0, nvr, hist, 0)

  def pfx(v, run):
    c = pos_smem[v]
    off_smem[v] = run
    pos_smem[v] = run
    return run + c

  lax.fori_loop(0, _NPW + 1, pfx, jnp.int32(0))

  nch = (cnt + _GCH - 1) // _GCH
  nfull = 2 * ((nch + 1) // 2)
  off_smem[_NPW + 1] = nfull * _GCH

  def scat(g, _):
    lvec = dlist[pl.ds(g * 16, 16)]
    svec = slist[pl.ds(g * 16, 16)]
    posv = jnp.zeros((16,), jnp.int32)
    for lane in range(16):
      ld = lvec[lane]
      p = pos_smem[ld]
      pos_smem[ld] = p + 1
      posv = jnp.where(iota == lane, jnp.full((16,), p, jnp.int32), posv)
    plsc.store_scatter(ssort, [posv], svec)
    return 0

  lax.fori_loop(0, nvr, scat, 0)
  return nfull


def _aggregate(h_hbm, ssort, off_smem, acc, rows0, rows1, sem0, sem1, nfull):
  neg = jnp.full((16,), -jnp.inf, jnp.float32)
  bufs = ((rows0, sem0), (rows1, sem1))

  def start(ci, b):
    rbuf, sem = bufs[b]
    pltpu.async_copy(h_hbm.at[ssort.at[pl.ds(ci * _GCH, _GCH)]], rbuf, sem)

  def wait(b):
    rbuf, sem = bufs[b]
    pltpu.make_async_copy(h_hbm.at[pl.ds(0, _GCH)], rbuf, sem).wait()

  def process(base, b, v0):
    rbuf, _ = bufs[b]
    bend = base + _GCH

    def cond(st):
      v, done = st
      return jnp.logical_not(done) & (off_smem[v] < bend)

    def nbody(st):
      v, _ = st
      beg = jnp.maximum(off_smem[v], base)
      end = jnp.minimum(off_smem[v + 1], bend)

      def eb(e, regs):
        return tuple(
            jnp.maximum(regs[j], rbuf[e - base, pl.ds(16 * j, 16)])
            for j in range(_D // 16))

      regs = lax.fori_loop(beg, end, eb, (neg,) * (_D // 16))
      for j in range(_D // 16):
        a = acc[v, pl.ds(16 * j, 16)]
        acc[v, pl.ds(16 * j, 16)] = jnp.maximum(a, regs[j])
      d = off_smem[v + 1] > bend
      return (jnp.where(d, v, v + 1), d)

    v, _ = lax.while_loop(cond, nbody, (v0, jnp.int32(0) > jnp.int32(1)))
    return v

  @pl.when(nfull > 0)
  def _():
    start(0, 0)

    def pair(p, v):
      c0 = 2 * p
      c1 = c0 + 1
      wait(0)
      start(c1, 1)
      v = process(c0 * _GCH, 0, v)
      wait(1)

      @pl.when(c1 + 1 < nfull)
      def _():
        start(c1 + 1, 0)

      return process(c1 * _GCH, 1, v)

    lax.fori_loop(0, nfull // 2, pair, jnp.int32(0))


def _finalize_and_store(acc, out_hbm, lo):
  big = jnp.float32(3.0e38)

  def body(i, _):
    for j in range(_D // 16):
      v = acc[i, pl.ds(16 * j, 16)]
      acc[i, pl.ds(16 * j, 16)] = jnp.where(jnp.abs(v) < big, v, 0.0)
    return 0

  lax.fori_loop(0, _NPW, body, 0)
  pltpu.sync_copy(acc.at[pl.ds(0, _NPW)], out_hbm.at[pl.ds(lo, _NPW)])


def _sc_layer1_body(src_hbm, dst_hbm, h_hbm,
                    out_hbm, slist_hbm, dlist_hbm, cnt_hbm,
                    sbuf0, dbuf0, sbuf1, dbuf1, slist, dlist, ssort, acc,
                    rows0, rows1, c16, pos_smem, off_smem,
                    sem0, sem1, ssem0, ssem1):
  wid = lax.axis_index("s") * _NC + lax.axis_index("c")
  lo = wid * _NPW
  hi = jnp.minimum(lo + _NPW, _N)

  _init_acc(acc)

  pad_s = jnp.full((16,), lo, jnp.int32)
  pad_d = jnp.full((16,), _NPW, jnp.int32)

  def prefill(i, _):
    slist[pl.ds(i * 16, 16)] = pad_s
    dlist[pl.ds(i * 16, 16)] = pad_d
    ssort[pl.ds(i * 16, 16)] = pad_s
    return 0

  lax.fori_loop(0, _CAP // 16, prefill, 0)

  lo_v = jnp.full((16,), lo, jnp.int32)
  hi_v = jnp.full((16,), hi, jnp.int32)
  sc_bufs = ((sbuf0, dbuf0, ssem0), (sbuf1, dbuf1, ssem1))

  def sstart(ci, b):
    sb, db, sem = sc_bufs[b]
    off = ci * _SCANCH
    pltpu.async_copy(src_hbm.at[pl.ds(off, _SCANCH)], sb, sem)
    pltpu.async_copy(dst_hbm.at[pl.ds(off, _SCANCH)], db, sem)

  def swait(b):
    sb, db, sem = sc_bufs[b]
    pltpu.make_async_copy(src_hbm.at[pl.ds(0, _SCANCH)], sb, sem).wait()
    pltpu.make_async_copy(src_hbm.at[pl.ds(0, _SCANCH)], db, sem).wait()

  def sscan(b, wp):
    sb, db, _ = sc_bufs[b]

    def scan_blk(o, wp):
      wp = jnp.minimum(wp, _CAP - 160 - 16)

      def scan16(i, wp):
        d16 = db[pl.ds(o * 160 + i * 16, 16)]
        s16 = sb[pl.ds(o * 160 + i * 16, 16)]
        m = (d16 >= lo_v) & (d16 < hi_v)
        c = plsc.all_reduce_population_count(m)[0]
        plsc.store_compressed(slist.at[pl.ds(wp, 16)], s16, mask=m)
        plsc.store_compressed(dlist.at[pl.ds(wp, 16)], d16 - lo_v, mask=m)
        return wp + c

      return lax.fori_loop(0, 10, scan16, wp, unroll=5)

    return lax.fori_loop(0, _SCANCH // 160, scan_blk, wp)

  sstart(0, 0)

  def scan_pair(p, wp):
    c1 = 2 * p + 1
    swait(0)
    sstart(c1, 1)
    wp = sscan(0, wp)
    swait(1)

    @pl.when(c1 + 1 < _NSCAN)
    def _():
      sstart(c1 + 1, 0)

    return sscan(1, wp)

  cnt = lax.fori_loop(0, _NSCAN // 2, scan_pair, jnp.int32(0))

  pltpu.sync_copy(slist, slist_hbm.at[wid])
  pltpu.sync_copy(dlist, dlist_hbm.at[wid])
  c16[...] = jnp.full((16,), cnt, jnp.int32)
  pltpu.sync_copy(c16, cnt_hbm.at[wid])

  nfull = _sort_edges(slist, dlist, ssort, pos_smem, off_smem, cnt)
  _aggregate(h_hbm, ssort, off_smem, acc, rows0, rows1, sem0, sem1, nfull)
  _finalize_and_store(acc, out_hbm, lo)


def _sc_layer2_body(slist_hbm, dlist_hbm, cnt_hbm, h_hbm,
                    out_hbm,
                    slist, dlist, ssort, acc, rows0, rows1, c16,
                    pos_smem, off_smem, sem0, sem1):
  wid = lax.axis_index("s") * _NC + lax.axis_index("c")
  lo = wid * _NPW

  _init_acc(acc)

  pad_s = jnp.full((16,), lo, jnp.int32)

  def prefill(i, _):
    ssort[pl.ds(i * 16, 16)] = pad_s
    return 0

  lax.fori_loop(0, _CAP // 16, prefill, 0)

  pltpu.sync_copy(slist_hbm.at[wid], slist)
  pltpu.sync_copy(dlist_hbm.at[wid], dlist)
  pltpu.sync_copy(cnt_hbm.at[wid], c16)
  cnt = c16[...][0]

  nfull = _sort_edges(slist, dlist, ssort, pos_smem, off_smem, cnt)
  _aggregate(h_hbm, ssort, off_smem, acc, rows0, rows1, sem0, sem1, nfull)
  _finalize_and_store(acc, out_hbm, lo)


def _sc_mesh():
  return plsc.VectorSubcoreMesh(core_axis_name="c", subcore_axis_name="s")


_sc_layer1 = pl.kernel(
    _sc_layer1_body,
    out_type=(
        jax.ShapeDtypeStruct((_NPAD, _D), jnp.float32),
        jax.ShapeDtypeStruct((_NW, _CAP), jnp.int32),
        jax.ShapeDtypeStruct((_NW, _CAP), jnp.int32),
        jax.ShapeDtypeStruct((_NW, 16), jnp.int32),
    ),
    mesh=_sc_mesh(),
    compiler_params=pltpu.CompilerParams(needs_layout_passes=False),
    scratch_types=(
        pltpu.VMEM((_SCANCH,), jnp.int32),
        pltpu.VMEM((_SCANCH,), jnp.int32),
        pltpu.VMEM((_SCANCH,), jnp.int32),
        pltpu.VMEM((_SCANCH,), jnp.int32),
        pltpu.VMEM((_CAP,), jnp.int32),
        pltpu.VMEM((_CAP,), jnp.int32),
        pltpu.VMEM((_CAP,), jnp.int32),
        pltpu.VMEM((_NACC, _D), jnp.float32),
        pltpu.VMEM((_GCH, _D), jnp.float32),
        pltpu.VMEM((_GCH, _D), jnp.float32),
        pltpu.VMEM((16,), jnp.int32),
        pltpu.SMEM((_NSM,), jnp.int32),
        pltpu.SMEM((_NSM,), jnp.int32),
        pltpu.SemaphoreType.DMA,
        pltpu.SemaphoreType.DMA,
        pltpu.SemaphoreType.DMA,
        pltpu.SemaphoreType.DMA,
    ),
)

_sc_layer2 = pl.kernel(
    _sc_layer2_body,
    out_type=jax.ShapeDtypeStruct((_NPAD, _D), jnp.float32),
    mesh=_sc_mesh(),
    compiler_params=pltpu.CompilerParams(needs_layout_passes=False),
    scratch_types=(
        pltpu.VMEM((_CAP,), jnp.int32),
        pltpu.VMEM((_CAP,), jnp.int32),
        pltpu.VMEM((_CAP,), jnp.int32),
        pltpu.VMEM((_NACC, _D), jnp.float32),
        pltpu.VMEM((_GCH, _D), jnp.float32),
        pltpu.VMEM((_GCH, _D), jnp.float32),
        pltpu.VMEM((16,), jnp.int32),
        pltpu.SMEM((_NSM,), jnp.int32),
        pltpu.SMEM((_NSM,), jnp.int32),
        pltpu.SemaphoreType.DMA,
        pltpu.SemaphoreType.DMA,
    ),
)


def kernel(x, edge_index, W1, b1, W2, b2,
           alpha0, scale0, shift0, alpha1, scale1, shift1,
           W_out, b_out):
  src = edge_index[0]
  dst = edge_index[1]
  w1 = W1.reshape(_C, _K)
  w2 = W2.reshape(_C, _K)
  b1r = b1.reshape(1, _C)
  b2r = b2.reshape(1, _C)
  al0 = alpha0.reshape(1, _D)
  sc0 = scale0.reshape(1, _D)
  sh0 = shift0.reshape(1, _D)
  al1 = alpha1.reshape(1, _D)
  sc1 = scale1.reshape(1, _D)
  sh1 = shift1.reshape(1, _D)

  h0 = _dense(x, al0, sc0, sh0, w1, b1r, relu=False)
  out0p, slist, dlist, cnts = _sc_layer1(src, dst, h0)
  h1 = _dense(out0p[:_N], al1, sc1, sh1, w2, b2r, relu=True)
  out1p = _sc_layer2(slist, dlist, cnts, h1)
  o = _readout(out1p[:_N], W_out, b_out.reshape(3, 1))
  return o[0:3, 0]
